# baseline (speedup 1.0000x reference)
import functools

import jax
import jax.numpy as jnp
from jax import lax
from jax.experimental import pallas as pl
from jax.experimental.pallas import tpu as pltpu
from jax.experimental.pallas import tpu_sc as plsc

N, E, D, ED, L = 10000, 320000, 128, 16, 4
NC, NS = 2, 16
NW = NC * NS
EPW = E // NW
CH = 80
NCH = EPW // CH
ZR = 24
NZ = 26
WPT = NZ * ZR
TAIL = N - NS * WPT
CW = 16
BN = 1000
BE = 1280

_f32 = jnp.float32


def _silu(v):
    return v * (1.0 / (1.0 + jnp.exp(-v)))


def _embed_body(x, win, binb, wp, bp, wq, h_o, p_o, q_o):
    h = jnp.dot(x[...], win[...], preferred_element_type=_f32) + binb[...]
    h_o[...] = h
    p_o[...] = jnp.dot(h, wp[...], preferred_element_type=_f32) + bp[...]
    q_o[...] = jnp.dot(h, wq[...], preferred_element_type=_f32)


def _edge_body(sa, sb, d16, ea, wr, we, ew2, eb2, cw1, cb1, cw2r,
               m_o, t_o):
    d = d16[...]
    radial = jnp.sum(d * d, axis=1, keepdims=True)
    pre1 = (sa[...] + sb[...] + radial * wr[...]
            + jnp.dot(ea[...], we[...], preferred_element_type=_f32))
    m1 = _silu(pre1)
    m = _silu(jnp.dot(m1, ew2[...], preferred_element_type=_f32) + eb2[...])
    t = _silu(jnp.dot(m, cw1[...], preferred_element_type=_f32) + cb1[...])
    s = jnp.sum(t * cw2r[...], axis=1, keepdims=True)
    one3 = (lax.broadcasted_iota(jnp.int32, (1, CW), 1) == 3).astype(_f32)
    m_o[...] = m
    t_o[...] = jnp.concatenate(
        [d * s + one3, jnp.zeros((d.shape[0], D - CW), _f32)], axis=1)


def _node_body(h, agg0, agg1, tpw, coord, nwa, nwb, nb1b, nw2, nb2b,
               wp, bp, wq, h_o, p_o, q_o, c_o, cw_o):
    agg = agg0[...] + agg1[...]
    hn = _silu(jnp.dot(h[...], nwa[...], preferred_element_type=_f32)
               + jnp.dot(agg, nwb[...], preferred_element_type=_f32)
               + nb1b[...])
    hn = jnp.dot(hn, nw2[...], preferred_element_type=_f32) + nb2b[...]
    hnew = h[...] + hn
    h_o[...] = hnew
    p_o[...] = jnp.dot(hnew, wp[...], preferred_element_type=_f32) + bp[...]
    q_o[...] = jnp.dot(hnew, wq[...], preferred_element_type=_f32)
    ts = jnp.sum(tpw[...], axis=0)
    cnt = jnp.maximum(ts[:, 3:4], 1.0)
    upd = jnp.concatenate(
        [ts[:, :3] / cnt, jnp.zeros((ts.shape[0], CW - 3), _f32)], axis=1)
    cnew = coord[...] + upd
    c_o[...] = cnew
    cw_o[...] = jnp.concatenate(
        [cnew, jnp.zeros((cnew.shape[0], D - CW), _f32)], axis=1)


def _wspec(w):
    return pl.BlockSpec(w.shape, lambda i: (0,) * w.ndim)


def _tc_embed(x, win, binb, wp, bp, wq):
    g = N // BN
    bs = lambda wdt: pl.BlockSpec((BN, wdt), lambda i: (i, 0))
    return pl.pallas_call(
        _embed_body,
        grid=(g,),
        in_specs=[bs(D)] + [_wspec(w) for w in (win, binb, wp, bp, wq)],
        out_specs=[bs(D), bs(D), bs(D)],
        out_shape=[jax.ShapeDtypeStruct((N, D), _f32)] * 3,
    )(x, win, binb, wp, bp, wq)


def _tc_edge(sa, sb, d16, ea, wr, we, ew2, eb2, cw1, cb1, cw2r):
    g = E // BE
    bs = lambda wdt: pl.BlockSpec((BE, wdt), lambda i: (i, 0))
    return pl.pallas_call(
        _edge_body,
        grid=(g,),
        in_specs=[bs(D), bs(D), bs(CW), bs(ED)]
        + [_wspec(w) for w in (wr, we, ew2, eb2, cw1, cb1, cw2r)],
        out_specs=[bs(D), bs(D)],
        out_shape=[jax.ShapeDtypeStruct((E, D), _f32),
                   jax.ShapeDtypeStruct((E, D), _f32)],
    )(sa, sb, d16, ea, wr, we, ew2, eb2, cw1, cb1, cw2r)


def _tc_node(h, agg0, agg1, tpw, coord, nwa, nwb, nb1b, nw2, nb2b,
             wp, bp, wq):
    g = N // BN
    bs = lambda wdt: pl.BlockSpec((BN, wdt), lambda i: (i, 0))
    tspec = pl.BlockSpec((NW, BN, 4), lambda i: (0, i, 0))
    return pl.pallas_call(
        _node_body,
        grid=(g,),
        in_specs=[bs(D), bs(D), bs(D), tspec, bs(CW)]
        + [_wspec(w) for w in (nwa, nwb, nb1b, nw2, nb2b, wp, bp, wq)],
        out_specs=[bs(D), bs(D), bs(D), bs(CW), bs(D)],
        out_shape=[jax.ShapeDtypeStruct((N, D), _f32)] * 3
        + [jax.ShapeDtypeStruct((N, CW), _f32),
           jax.ShapeDtypeStruct((N, D), _f32)],
    )(h, agg0, agg1, tpw, coord, nwa, nwb, nb1b, nw2, nb2b, wp, bp, wq)


@functools.cache
def _sc_gather_call():
    mesh = plsc.VectorSubcoreMesh(core_axis_name="c", subcore_axis_name="s")
    return pl.kernel(
        _sc_gather_body,
        out_type=(jax.ShapeDtypeStruct((E, D), _f32),
                  jax.ShapeDtypeStruct((E, D), _f32),
                  jax.ShapeDtypeStruct((E, CW), _f32)),
        mesh=mesh,
        scratch_types=(
            pltpu.VMEM((NCH, CH), jnp.int32),
            pltpu.VMEM((NCH, CH), jnp.int32),
            pltpu.VMEM((CH, D), _f32),
            pltpu.VMEM((CH, D), _f32),
            pltpu.VMEM((CH, D), _f32),
            pltpu.VMEM((CH, D), _f32),
            pltpu.VMEM((CH, CW), _f32),
        ),
    )


def _sc_gather_body(p_t, q_t, c_t, rowf, colf, sa_o, sb_o, d_o,
                    idxr, idxc, bufa, bufb, bufr, bufc, dbuf):
    c = lax.axis_index("c")
    s = lax.axis_index("s")
    wid = c * NS + s
    base = wid * EPW
    pltpu.sync_copy(rowf.at[wid], idxr)
    pltpu.sync_copy(colf.at[wid], idxc)

    def chunk(ch, _):
        eoff = base + ch * CH
        pltpu.sync_copy(p_t.at[idxr.at[ch]], bufa)
        pltpu.sync_copy(q_t.at[idxc.at[ch]], bufb)
        pltpu.sync_copy(c_t.at[idxr.at[ch]], bufr)
        pltpu.sync_copy(c_t.at[idxc.at[ch]], bufc)

        def drow(r, _):
            dbuf[r, pl.ds(0, CW)] = (bufr[r, pl.ds(0, CW)]
                                     - bufc[r, pl.ds(0, CW)])
            return 0

        lax.fori_loop(0, CH, drow, 0)
        pltpu.sync_copy(bufa, sa_o.at[pl.ds(eoff, CH)])
        pltpu.sync_copy(bufb, sb_o.at[pl.ds(eoff, CH)])
        pltpu.sync_copy(dbuf, d_o.at[pl.ds(eoff, CH)])
        return 0

    lax.fori_loop(0, NCH, chunk, 0)


@functools.cache
def _sc_scatter_m_call():
    mesh = plsc.VectorSubcoreMesh(core_axis_name="c", subcore_axis_name="s")
    return pl.kernel(
        _sc_scatter_m_body,
        out_type=jax.ShapeDtypeStruct((NC, N, D), _f32),
        mesh=mesh,
        scratch_types=(
            pltpu.VMEM((NCH, CH), jnp.int32),
            pltpu.VMEM((CH, D), _f32),
            pltpu.VMEM((CH, D), _f32),
            pltpu.VMEM((ZR, D), _f32),
            pltpu.VMEM_SHARED((N, D), _f32),
            pltpu.SemaphoreType.DMA,
            pltpu.SemaphoreType.DMA,
            pltpu.SemaphoreType.DMA,
        ),
    )


def _sc_scatter_m_body(m_t, rowf, agg_o, idxv, mb0, mb1, zbig,
                       agg_s, rs0, rs1, ssem):
    c = lax.axis_index("c")
    s = lax.axis_index("s")
    wid = c * NS + s
    base = wid * EPW

    def zrow(r, _):
        for j in range(D // 16):
            zbig[r, pl.ds(j * 16, 16)] = jnp.zeros((16,), _f32)
        return 0

    lax.fori_loop(0, ZR, zrow, 0)

    for k in range(NZ):
        pltpu.sync_copy(zbig, agg_s.at[pl.ds(s * WPT + k * ZR, ZR)])

    @pl.when(s == 0)
    def _():
        pltpu.sync_copy(zbig.at[pl.ds(0, TAIL)], agg_s.at[pl.ds(NS * WPT, TAIL)])

    plsc.subcore_barrier()
    pltpu.sync_copy(rowf.at[wid], idxv)

    bufs = (mb0, mb1)
    rsems = (rs0, rs1)
    nb = 2

    def issue_read(ch, b):
        pltpu.async_copy(m_t.at[pl.ds(base + ch * CH, CH)], bufs[b], rsems[b])

    def wait_read(b):
        pltpu.make_async_copy(m_t.at[pl.ds(base, CH)], bufs[b], rsems[b]).wait()

    for b in range(nb):
        issue_read(b, b)

    def group(g, _):
        for b in range(nb):
            ch = g * nb + b
            wait_read(b)
            pltpu.async_copy(bufs[b], agg_s.at[idxv.at[ch]], ssem, add=True)
            pltpu.make_async_copy(bufs[b], agg_s.at[idxv.at[ch]], ssem).wait()

            @pl.when(ch + nb < NCH)
            def _():
                issue_read(ch + nb, b)

        return 0

    lax.fori_loop(0, NCH // nb, group, 0)
    for ch in range((NCH // nb) * nb, NCH):
        b = ch % nb
        wait_read(b)
        pltpu.async_copy(bufs[b], agg_s.at[idxv.at[ch]], ssem, add=True)
        pltpu.make_async_copy(bufs[b], agg_s.at[idxv.at[ch]], ssem).wait()
    plsc.subcore_barrier()

    for k in range(NZ):
        sl = pl.ds(s * WPT + k * ZR, ZR)
        pltpu.sync_copy(agg_s.at[sl], agg_o.at[c, sl])

    @pl.when(s == 0)
    def _():
        tl = pl.ds(NS * WPT, TAIL)
        pltpu.sync_copy(agg_s.at[tl], agg_o.at[c, tl])


TAR = 320


@functools.cache
def _sc_scatter_t_call():
    mesh = plsc.VectorSubcoreMesh(core_axis_name="c", subcore_axis_name="s")
    return pl.kernel(
        _sc_scatter_t_body,
        out_type=jax.ShapeDtypeStruct((NW * TAR * 128,), _f32),
        mesh=mesh,
        scratch_types=(
            pltpu.VMEM((NCH, CH), jnp.int32),
            pltpu.VMEM((CH, D), _f32),
            pltpu.VMEM((CH, D), _f32),
            pltpu.VMEM((TAR * 128,), _f32),
            pltpu.SemaphoreType.DMA,
            pltpu.SemaphoreType.DMA,
        ),
    )


def _sc_scatter_t_body(t_t, rowf, tp_o, idxv, tb0, tb1, tacc, rs0, rs1):
    c = lax.axis_index("c")
    s = lax.axis_index("s")
    wid = c * NS + s
    base = wid * EPW

    def zrow(r, _):
        tacc[pl.ds(r * 16, 16)] = jnp.zeros((16,), _f32)
        return 0

    lax.fori_loop(0, TAR * 8, zrow, 0)
    pltpu.sync_copy(rowf.at[wid], idxv)

    bufs = (tb0, tb1)
    rsems = (rs0, rs1)

    def issue_read(ch, b):
        pltpu.async_copy(t_t.at[pl.ds(base + ch * CH, CH)], bufs[b], rsems[b])

    def wait_read(b):
        pltpu.make_async_copy(t_t.at[pl.ds(base, CH)], bufs[b], rsems[b]).wait()

    def do_edges(ch, buf_ref):
        def eg(g16, _):
            iv = idxv[ch, pl.ds(g16 * 16, 16)]
            for k in range(16):
                off = iv[k] * 4
                vals = buf_ref[g16 * 16 + k, pl.ds(0, 16)]
                tacc[pl.ds(off, 16)] = tacc[pl.ds(off, 16)] + vals
            return 0

        lax.fori_loop(0, CH // 16, eg, 0)

    issue_read(0, 0)
    issue_read(1, 1)

    def group(g2, _):
        for b in range(2):
            ch = g2 * 2 + b
            wait_read(b)
            do_edges(ch, bufs[b])

            @pl.when(ch + 2 < NCH)
            def _():
                issue_read(ch + 2, b)

        return 0

    lax.fori_loop(0, NCH // 2, group, 0)
    for ch in range((NCH // 2) * 2, NCH):
        b = ch % 2
        wait_read(b)
        do_edges(ch, bufs[b])
    pltpu.sync_copy(tacc, tp_o.at[pl.ds(wid * TAR * 128, TAR * 128)])


def kernel(x, pos, edge_index, edge_attr, emb_in_w, emb_in_b, emb_out_w,
           emb_out_b, ew1, eb1, ew2, eb2, nw1, nb1, nw2, nb2, cw1, cb1, cw2):
    row = edge_index[0].astype(jnp.int32)
    col = edge_index[1].astype(jnp.int32)
    rowf = row.reshape(NW, NCH, CH)
    colf = col.reshape(NW, NCH, CH)
    coord = jnp.pad(pos, ((0, 0), (0, CW - 3)))

    wa = [ew1[l, :D] for l in range(L)]
    wb = [ew1[l, D:2 * D] for l in range(L)]
    wr = [ew1[l, 2 * D:2 * D + 1] for l in range(L)]
    we = [ew1[l, 2 * D + 1:] for l in range(L)]
    eb1b = [eb1[l].reshape(1, D) for l in range(L)]
    eb2b = [eb2[l].reshape(1, D) for l in range(L)]
    nwa = [nw1[l, :D] for l in range(L)]
    nwb = [nw1[l, D:] for l in range(L)]
    nb1b = [nb1[l].reshape(1, D) for l in range(L)]
    nb2b = [nb2[l].reshape(1, D) for l in range(L)]
    cb1b = [cb1[l].reshape(1, D) for l in range(L)]
    cw2r = [cw2[l].reshape(1, D) for l in range(L)]

    cwide = jnp.pad(pos, ((0, 0), (0, D - 3)))
    h, p, q = _tc_embed(x, emb_in_w, emb_in_b.reshape(1, D),
                        wa[0], eb1b[0], wb[0])
    for l in range(L):
        sa, sb, d16 = _sc_gather_call()(p, q, cwide, rowf, colf)
        m, t16 = _tc_edge(sa, sb, d16, edge_attr, wr[l], we[l], ew2[l],
                          eb2b[l], cw1[l], cb1b[l], cw2r[l])
        aggp = _sc_scatter_m_call()(m, rowf)
        tpr = _sc_scatter_t_call()(t16, rowf)
        tpw = tpr.reshape(NW, TAR * 128)[:, :N * 4].reshape(NW, N, 4)

        if l + 1 < L:
            wp, bp, wq = wa[l + 1], eb1b[l + 1], wb[l + 1]
        else:
            wp, bp, wq = emb_out_w, emb_out_b.reshape(1, D), emb_out_w
        h, p, q, coord, cwide = _tc_node(
            h, aggp[0], aggp[1], tpw, coord,
            nwa[l], nwb[l], nb1b[l], nw2[l], nb2b[l], wp, bp, wq)
    return p, coord[:, :3]


# speedup vs baseline: 1.3282x; 1.3282x over previous
import functools

import jax
import jax.numpy as jnp
from jax import lax
from jax.experimental import pallas as pl
from jax.experimental.pallas import tpu as pltpu
from jax.experimental.pallas import tpu_sc as plsc

N, E, D, ED, L = 10000, 320000, 128, 16, 4
NC, NS = 2, 16
NW = NC * NS
EPW = E // NW
CH = 80
NCH = EPW // CH
ZR = 24
NZ = 26
WPT = NZ * ZR
TAIL = N - NS * WPT
CW = 16
BN = 1000
BE = 1280

_f32 = jnp.float32


def _silu(v):
    return v * (1.0 / (1.0 + jnp.exp(-v)))


def _embed_body(x, win, binb, wp, bp, wq, h_o, p_o, q_o):
    h = jnp.dot(x[...], win[...], preferred_element_type=_f32) + binb[...]
    h_o[...] = h
    p_o[...] = jnp.dot(h, wp[...], preferred_element_type=_f32) + bp[...]
    q_o[...] = jnp.dot(h, wq[...], preferred_element_type=_f32)


def _edge_body(sa, sb, d16, ea, wr, we, ew2, eb2, cw1, cb1, cw2r,
               m_o, t_o):
    d = d16[...]
    radial = jnp.sum(d * d, axis=1, keepdims=True)
    pre1 = (sa[...] + sb[...] + radial * wr[...]
            + jnp.dot(ea[...], we[...], preferred_element_type=_f32))
    m1 = _silu(pre1)
    m = _silu(jnp.dot(m1, ew2[...], preferred_element_type=_f32) + eb2[...])
    t = _silu(jnp.dot(m, cw1[...], preferred_element_type=_f32) + cb1[...])
    s = jnp.sum(t * cw2r[...], axis=1, keepdims=True)
    one3 = (lax.broadcasted_iota(jnp.int32, (1, CW), 1) == 3).astype(_f32)
    m_o[...] = m
    t_o[...] = d * s + one3


def _node_body(h, agg0, agg1, tpw, coord, nwa, nwb, nb1b, nw2, nb2b,
               wp, bp, wq, h_o, p_o, q_o, c_o):
    agg = agg0[...] + agg1[...]
    hn = _silu(jnp.dot(h[...], nwa[...], preferred_element_type=_f32)
               + jnp.dot(agg, nwb[...], preferred_element_type=_f32)
               + nb1b[...])
    hn = jnp.dot(hn, nw2[...], preferred_element_type=_f32) + nb2b[...]
    hnew = h[...] + hn
    h_o[...] = hnew
    p_o[...] = jnp.dot(hnew, wp[...], preferred_element_type=_f32) + bp[...]
    q_o[...] = jnp.dot(hnew, wq[...], preferred_element_type=_f32)
    ts = jnp.sum(tpw[...], axis=0)
    cnt = jnp.maximum(ts[:, 3:4], 1.0)
    upd = jnp.concatenate(
        [ts[:, :3] / cnt, jnp.zeros((ts.shape[0], CW - 3), _f32)], axis=1)
    cnew = coord[...] + upd
    c_o[...] = cnew


def _wspec(w):
    return pl.BlockSpec(w.shape, lambda i: (0,) * w.ndim)


def _tc_embed(x, win, binb, wp, bp, wq):
    g = N // BN
    bs = lambda wdt: pl.BlockSpec((BN, wdt), lambda i: (i, 0))
    return pl.pallas_call(
        _embed_body,
        grid=(g,),
        in_specs=[bs(D)] + [_wspec(w) for w in (win, binb, wp, bp, wq)],
        out_specs=[bs(D), bs(D), bs(D)],
        out_shape=[jax.ShapeDtypeStruct((N, D), _f32)] * 3,
    )(x, win, binb, wp, bp, wq)


def _tc_edge(sa, sb, d16, ea, wr, we, ew2, eb2, cw1, cb1, cw2r):
    g = E // BE
    bs = lambda wdt: pl.BlockSpec((BE, wdt), lambda i: (i, 0))
    return pl.pallas_call(
        _edge_body,
        grid=(g,),
        in_specs=[bs(D), bs(D), bs(CW), bs(ED)]
        + [_wspec(w) for w in (wr, we, ew2, eb2, cw1, cb1, cw2r)],
        out_specs=[bs(D), bs(CW)],
        out_shape=[jax.ShapeDtypeStruct((E, D), _f32),
                   jax.ShapeDtypeStruct((E, CW), _f32)],
    )(sa, sb, d16, ea, wr, we, ew2, eb2, cw1, cb1, cw2r)


def _tc_node(h, agg0, agg1, tpw, coord, nwa, nwb, nb1b, nw2, nb2b,
             wp, bp, wq):
    g = N // BN
    bs = lambda wdt: pl.BlockSpec((BN, wdt), lambda i: (i, 0))
    tspec = pl.BlockSpec((NW, BN, 4), lambda i: (0, i, 0))
    return pl.pallas_call(
        _node_body,
        grid=(g,),
        in_specs=[bs(D), bs(D), bs(D), tspec, bs(CW)]
        + [_wspec(w) for w in (nwa, nwb, nb1b, nw2, nb2b, wp, bp, wq)],
        out_specs=[bs(D), bs(D), bs(D), bs(CW)],
        out_shape=[jax.ShapeDtypeStruct((N, D), _f32)] * 3
        + [jax.ShapeDtypeStruct((N, CW), _f32)],
    )(h, agg0, agg1, tpw, coord, nwa, nwb, nb1b, nw2, nb2b, wp, bp, wq)


@functools.cache
def _sc_gather_call():
    mesh = plsc.VectorSubcoreMesh(core_axis_name="c", subcore_axis_name="s")
    return pl.kernel(
        _sc_gather_body,
        out_type=(jax.ShapeDtypeStruct((E, D), _f32),
                  jax.ShapeDtypeStruct((E, D), _f32),
                  jax.ShapeDtypeStruct((E, CW), _f32)),
        mesh=mesh,
        scratch_types=(
            pltpu.VMEM((EPW,), jnp.int32),
            pltpu.VMEM((EPW,), jnp.int32),
            pltpu.VMEM((CH, D), _f32),
            pltpu.VMEM((CH, D), _f32),
            pltpu.VMEM((CH, D), _f32),
            pltpu.VMEM((CH, D), _f32),
            pltpu.VMEM((CH, D), _f32),
            pltpu.VMEM((CH, D), _f32),
            pltpu.VMEM((CH, D), _f32),
            pltpu.VMEM((CH, D), _f32),
            pltpu.VMEM((CH, CW), _f32),
            pltpu.SemaphoreType.DMA,
            pltpu.SemaphoreType.DMA,
            pltpu.SemaphoreType.DMA,
            pltpu.SemaphoreType.DMA,
            pltpu.SemaphoreType.DMA,
        ),
    )


def _sc_gather_body(p_t, q_t, c_t, rowfl, colfl, sa_o, sb_o, d_o,
                    idxr, idxc, ba0, ba1, bb0, bb1, br0, br1, bc0, bc1,
                    dbuf, gs0, gs1, ws0, ws1, dsem):
    c = lax.axis_index("c")
    s = lax.axis_index("s")
    wid = c * NS + s
    base = wid * EPW
    pltpu.sync_copy(rowfl.at[pl.ds(base, EPW)], idxr)
    pltpu.sync_copy(colfl.at[pl.ds(base, EPW)], idxc)

    bas, bbs = (ba0, ba1), (bb0, bb1)
    brs, bcs = (br0, br1), (bc0, bc1)
    gsems, wsems = (gs0, gs1), (ws0, ws1)

    def issue_g(ch, b):
        ir = idxr.at[pl.ds(ch * CH, CH)]
        ic = idxc.at[pl.ds(ch * CH, CH)]
        pltpu.async_copy(p_t.at[ir], bas[b], gsems[b])
        pltpu.async_copy(q_t.at[ic], bbs[b], gsems[b])
        pltpu.async_copy(c_t.at[ir], brs[b], gsems[b])
        pltpu.async_copy(c_t.at[ic], bcs[b], gsems[b])

    def wait_g(ch, b):
        ir = idxr.at[pl.ds(0, CH)]
        pltpu.make_async_copy(p_t.at[ir], bas[b], gsems[b]).wait()
        pltpu.make_async_copy(p_t.at[ir], bbs[b], gsems[b]).wait()
        pltpu.make_async_copy(c_t.at[ir], brs[b], gsems[b]).wait()
        pltpu.make_async_copy(c_t.at[ir], bcs[b], gsems[b]).wait()

    def issue_w(ch, b):
        eoff = base + ch * CH
        pltpu.async_copy(bas[b], sa_o.at[pl.ds(eoff, CH)], wsems[b])
        pltpu.async_copy(bbs[b], sb_o.at[pl.ds(eoff, CH)], wsems[b])
        pltpu.async_copy(dbuf, d_o.at[pl.ds(eoff, CH)], dsem)

    def wait_w(b):
        pltpu.make_async_copy(bas[b], sa_o.at[pl.ds(base, CH)], wsems[b]).wait()
        pltpu.make_async_copy(bbs[b], sb_o.at[pl.ds(base, CH)], wsems[b]).wait()

    def wait_d():
        pltpu.make_async_copy(dbuf, d_o.at[pl.ds(base, CH)], dsem).wait()

    def process(ch, b):
        wait_g(ch, b)

        @pl.when(ch >= 1)
        def _():
            wait_d()

        def drow(r, _):
            dbuf[r, pl.ds(0, CW)] = (brs[b][r, pl.ds(0, CW)]
                                     - bcs[b][r, pl.ds(0, CW)])
            return 0

        lax.fori_loop(0, CH, drow, 0)
        issue_w(ch, b)

    issue_g(0, 0)

    def pair(g2, _):
        for b in range(2):
            ch = g2 * 2 + b
            q = 1 - b

            @pl.when(ch + 1 < NCH)
            def _():
                @pl.when(ch >= 1)
                def _():
                    wait_w(q)

                issue_g(ch + 1, q)

            process(ch, b)
        return 0

    lax.fori_loop(0, NCH // 2, pair, 0)
    for ch in range((NCH // 2) * 2, NCH):
        b = ch % 2
        wait_w(1 - b)
        process(ch, b)
    wait_w((NCH - 1) % 2)
    wait_d()


@functools.cache
def _sc_scatter_m_call():
    mesh = plsc.VectorSubcoreMesh(core_axis_name="c", subcore_axis_name="s")
    return pl.kernel(
        _sc_scatter_m_body,
        out_type=jax.ShapeDtypeStruct((NC, N, D), _f32),
        mesh=mesh,
        scratch_types=(
            pltpu.VMEM((NCH, CH), jnp.int32),
            pltpu.VMEM((CH, D), _f32),
            pltpu.VMEM((CH, D), _f32),
            pltpu.VMEM((ZR, D), _f32),
            pltpu.VMEM_SHARED((N, D), _f32),
            pltpu.SemaphoreType.DMA,
            pltpu.SemaphoreType.DMA,
            pltpu.SemaphoreType.DMA,
        ),
    )


def _sc_scatter_m_body(m_t, rowf, agg_o, idxv, mb0, mb1, zbig,
                       agg_s, rs0, rs1, ssem):
    c = lax.axis_index("c")
    s = lax.axis_index("s")
    wid = c * NS + s
    base = wid * EPW

    def zrow(r, _):
        for j in range(D // 16):
            zbig[r, pl.ds(j * 16, 16)] = jnp.zeros((16,), _f32)
        return 0

    lax.fori_loop(0, ZR, zrow, 0)

    for k in range(NZ):
        pltpu.sync_copy(zbig, agg_s.at[pl.ds(s * WPT + k * ZR, ZR)])

    @pl.when(s == 0)
    def _():
        pltpu.sync_copy(zbig.at[pl.ds(0, TAIL)], agg_s.at[pl.ds(NS * WPT, TAIL)])

    plsc.subcore_barrier()
    pltpu.sync_copy(rowf.at[wid], idxv)

    bufs = (mb0, mb1)
    rsems = (rs0, rs1)
    nb = 2

    def issue_read(ch, b):
        pltpu.async_copy(m_t.at[pl.ds(base + ch * CH, CH)], bufs[b], rsems[b])

    def wait_read(b):
        pltpu.make_async_copy(m_t.at[pl.ds(base, CH)], bufs[b], rsems[b]).wait()

    for b in range(nb):
        issue_read(b, b)

    def group(g, _):
        for b in range(nb):
            ch = g * nb + b
            wait_read(b)
            pltpu.async_copy(bufs[b], agg_s.at[idxv.at[ch]], ssem, add=True)
            pltpu.make_async_copy(bufs[b], agg_s.at[idxv.at[ch]], ssem).wait()

            @pl.when(ch + nb < NCH)
            def _():
                issue_read(ch + nb, b)

        return 0

    lax.fori_loop(0, NCH // nb, group, 0)
    for ch in range((NCH // nb) * nb, NCH):
        b = ch % nb
        wait_read(b)
        pltpu.async_copy(bufs[b], agg_s.at[idxv.at[ch]], ssem, add=True)
        pltpu.make_async_copy(bufs[b], agg_s.at[idxv.at[ch]], ssem).wait()
    plsc.subcore_barrier()

    for k in range(NZ):
        sl = pl.ds(s * WPT + k * ZR, ZR)
        pltpu.sync_copy(agg_s.at[sl], agg_o.at[c, sl])

    @pl.when(s == 0)
    def _():
        tl = pl.ds(NS * WPT, TAIL)
        pltpu.sync_copy(agg_s.at[tl], agg_o.at[c, tl])


TAR = 320


@functools.cache
def _sc_scatter_t_call():
    mesh = plsc.VectorSubcoreMesh(core_axis_name="c", subcore_axis_name="s")
    return pl.kernel(
        _sc_scatter_t_body,
        out_type=jax.ShapeDtypeStruct((NW * TAR * 128,), _f32),
        mesh=mesh,
        scratch_types=(
            pltpu.VMEM((NCH, CH), jnp.int32),
            pltpu.VMEM((CH, CW), _f32),
            pltpu.VMEM((CH, CW), _f32),
            pltpu.VMEM((TAR * 128,), _f32),
            pltpu.SemaphoreType.DMA,
            pltpu.SemaphoreType.DMA,
        ),
    )


def _sc_scatter_t_body(t_t, rowf, tp_o, idxv, tb0, tb1, tacc, rs0, rs1):
    c = lax.axis_index("c")
    s = lax.axis_index("s")
    wid = c * NS + s
    base = wid * EPW

    def zrow(r, _):
        tacc[pl.ds(r * 16, 16)] = jnp.zeros((16,), _f32)
        return 0

    lax.fori_loop(0, TAR * 8, zrow, 0)
    pltpu.sync_copy(rowf.at[wid], idxv)

    bufs = (tb0, tb1)
    rsems = (rs0, rs1)

    def issue_read(ch, b):
        pltpu.async_copy(t_t.at[pl.ds(base + ch * CH, CH)], bufs[b], rsems[b])

    def wait_read(b):
        pltpu.make_async_copy(t_t.at[pl.ds(base, CH)], bufs[b], rsems[b]).wait()

    def do_edges(ch, buf_ref):
        def eg(g16, _):
            iv = idxv[ch, pl.ds(g16 * 16, 16)]
            for k in range(16):
                off = iv[k] * 4
                vals = buf_ref[g16 * 16 + k, pl.ds(0, 16)]
                tacc[pl.ds(off, 16)] = tacc[pl.ds(off, 16)] + vals
            return 0

        lax.fori_loop(0, CH // 16, eg, 0)

    issue_read(0, 0)
    issue_read(1, 1)

    def group(g2, _):
        for b in range(2):
            ch = g2 * 2 + b
            wait_read(b)
            do_edges(ch, bufs[b])

            @pl.when(ch + 2 < NCH)
            def _():
                issue_read(ch + 2, b)

        return 0

    lax.fori_loop(0, NCH // 2, group, 0)
    for ch in range((NCH // 2) * 2, NCH):
        b = ch % 2
        wait_read(b)
        do_edges(ch, bufs[b])
    pltpu.sync_copy(tacc, tp_o.at[pl.ds(wid * TAR * 128, TAR * 128)])


def kernel(x, pos, edge_index, edge_attr, emb_in_w, emb_in_b, emb_out_w,
           emb_out_b, ew1, eb1, ew2, eb2, nw1, nb1, nw2, nb2, cw1, cb1, cw2):
    row = edge_index[0].astype(jnp.int32)
    col = edge_index[1].astype(jnp.int32)
    rowf = row.reshape(NW, NCH, CH)
    colf = col.reshape(NW, NCH, CH)
    coord = jnp.pad(pos, ((0, 0), (0, CW - 3)))

    rowl = row
    coll = col

    def ctable(c16):
        return jnp.pad(c16, ((0, 0), (0, D - CW)))

    wa = [ew1[l, :D] for l in range(L)]
    wb = [ew1[l, D:2 * D] for l in range(L)]
    wr = [ew1[l, 2 * D:2 * D + 1] for l in range(L)]
    we = [ew1[l, 2 * D + 1:] for l in range(L)]
    eb1b = [eb1[l].reshape(1, D) for l in range(L)]
    eb2b = [eb2[l].reshape(1, D) for l in range(L)]
    nwa = [nw1[l, :D] for l in range(L)]
    nwb = [nw1[l, D:] for l in range(L)]
    nb1b = [nb1[l].reshape(1, D) for l in range(L)]
    nb2b = [nb2[l].reshape(1, D) for l in range(L)]
    cb1b = [cb1[l].reshape(1, D) for l in range(L)]
    cw2r = [cw2[l].reshape(1, D) for l in range(L)]

    h, p, q = _tc_embed(x, emb_in_w, emb_in_b.reshape(1, D),
                        wa[0], eb1b[0], wb[0])
    for l in range(L):
        sa, sb, d16 = _sc_gather_call()(p, q, ctable(coord), rowl, coll)
        m, t16 = _tc_edge(sa, sb, d16, edge_attr, wr[l], we[l], ew2[l],
                          eb2b[l], cw1[l], cb1b[l], cw2r[l])
        aggp = _sc_scatter_m_call()(m, rowf)
        tpr = _sc_scatter_t_call()(t16, rowf)
        tpw = tpr.reshape(NW, TAR * 128)[:, :N * 4].reshape(NW, N, 4)

        if l + 1 < L:
            wp, bp, wq = wa[l + 1], eb1b[l + 1], wb[l + 1]
        else:
            wp, bp, wq = emb_out_w, emb_out_b.reshape(1, D), emb_out_w
        h, p, q, coord = _tc_node(
            h, aggp[0], aggp[1], tpw, coord,
            nwa[l], nwb[l], nb1b[l], nw2[l], nb2b[l], wp, bp, wq)
    return p, coord[:, :3]
